# baseline (speedup 1.0000x reference)
import functools

import jax
import jax.numpy as jnp
from jax import lax
from jax.experimental import pallas as pl
from jax.experimental.pallas import tpu as pltpu
from jax.experimental.pallas import tpu_sc as plsc

_W_AREA = 0.1
_W_ASPECT = 0.1
_W_DIHEDRAL = 0.05
_MIN_AREA = 1e-06
_MAX_AR = 10.0

_L = 16
_NCORE = 2
_NS = 16
_NT = _NCORE * _NS
_SUB = 320
_NCP = 16
_TCB = 2048


def _rsqrt(x):
    i = lax.bitcast_convert_type(x, jnp.int32)
    i = 0x5F3759DF - lax.shift_right_logical(i, 1)
    y = lax.bitcast_convert_type(i, jnp.float32)
    for _ in range(3):
        y = y * (1.5 - 0.5 * x * y * y)
    return y


def _sqrt(x):
    return x * _rsqrt(jnp.maximum(x, 1e-36))


def _make_sc_call(V, B, F, F_pad):
    PT = F_pad // _NT
    NSUBS = PT // _SUB
    NC = B * 3
    RUN = 3 * _SUB
    TVP = -(-(3 * V * B) // (12 * 128)) * 12 * 128

    mesh = plsc.VectorSubcoreMesh(
        core_axis_name="c", subcore_axis_name="s", num_cores=_NCORE,
        num_subcores=_NS)

    def body(*refs):
        tab_hbm = refs[0]
        f_hbm = refs[1:4]
        nrm_hbm = refs[4]
        parts_hbm = refs[5]
        sc = refs[6:]
        idxall_v = sc[0]
        idx3_v = sc[1:10]
        comp_v = sc[10:10 + NC]
        nstage_v = sc[10 + NC:10 + 2 * NC]
        mypart_v = sc[10 + 2 * NC]
        tab_s = sc[11 + 2 * NC]
        sem = sc[12 + 2 * NC]
        sem2 = sc[13 + 2 * NC]

        iota = lax.iota(jnp.int32, _L)
        sid = lax.axis_index("s")
        cid = lax.axis_index("c")
        gtile = sid * _NCORE + cid
        tile_base = gtile * PT

        TW = TVP // 12
        for q in range(12):
            @pl.when(sid == q)
            def _():
                pltpu.sync_copy(tab_hbm.at[pl.ds(q * TW, TW)],
                                tab_s.at[pl.ds(q * TW, TW)])
        for s in range(3):
            pltpu.sync_copy(f_hbm[s].at[pl.ds(tile_base, PT)],
                            idxall_v.at[pl.ds(s * PT, PT)])
        plsc.subcore_barrier()

        def p1_body(t, carry):
            accs = list(carry)
            sub_base = tile_base + t * _SUB
            for s in range(3):
                for g in range(_SUB // _L):
                    iv = idxall_v[pl.ds(s * PT + t * _SUB + g * _L, _L)]
                    i3 = iv * 3
                    for c in range(3):
                        idx3_v[s * 3 + c][pl.ds(g * _L, _L)] = i3 + c
            cps = [pltpu.async_copy(
                tab_s.at[pl.ds(b * 3 * V, 3 * V)].at[idx3_v[s * 3 + c]],
                comp_v[3 * b + c].at[pl.ds(s * _SUB, _SUB)], sem)
                for b in range(B) for c in range(3) for s in range(3)]
            for cp in cps:
                cp.wait()

            for g in range(_SUB // _L):
                jbase = g * _L
                gidx = sub_base + jbase + iota
                mask = jnp.where(gidx < F, 1.0, 0.0)
                for b in range(B):
                    c0 = 3 * b
                    (v0x, v0y, v0z), (v1x, v1y, v1z), (v2x, v2y, v2z) = [
                        [comp_v[c0 + c][pl.ds(s * _SUB + jbase, _L)]
                         for c in range(3)] for s in range(3)]
                    ax, ay, az = v1x - v0x, v1y - v0y, v1z - v0z
                    bx, by, bz = v2x - v0x, v2y - v0y, v2z - v0z
                    cx = ay * bz - az * by
                    cy = az * bx - ax * bz
                    cz = ax * by - ay * bx
                    cn2 = cx * cx + cy * cy + cz * cz
                    r = _rsqrt(jnp.maximum(cn2, 1e-36))
                    s_cn = cn2 * r
                    accs[12] = accs[12] + mask * jnp.maximum(
                        _MIN_AREA - 0.5 * s_cn, 0.0)
                    f = jnp.where(s_cn >= 1e-8, r, 1e8)
                    nx, ny, nz = cx * f, cy * f, cz * f
                    accs[c0 + 0] = accs[c0 + 0] + mask * nx
                    accs[c0 + 1] = accs[c0 + 1] + mask * ny
                    accs[c0 + 2] = accs[c0 + 2] + mask * nz
                    nstage_v[c0 + 0][pl.ds(jbase, _L)] = nx
                    nstage_v[c0 + 1][pl.ds(jbase, _L)] = ny
                    nstage_v[c0 + 2][pl.ds(jbase, _L)] = nz
                    l1 = ax * ax + ay * ay + az * az
                    gx, gy, gz = v2x - v1x, v2y - v1y, v2z - v1z
                    l2 = gx * gx + gy * gy + gz * gz
                    l3 = bx * bx + by * by + bz * bz
                    mx2 = jnp.maximum(jnp.maximum(l1, l2), l3)
                    mn2 = jnp.minimum(jnp.minimum(l1, l2), l3)
                    ar = _sqrt(mx2 / jnp.maximum(mn2, 1e-16))
                    accs[13] = accs[13] + mask * jnp.maximum(
                        ar - _MAX_AR, 0.0)
            wcps = [pltpu.async_copy(
                nstage_v[q], nrm_hbm.at[pl.ds(q * F_pad + sub_base, _SUB)],
                sem2) for q in range(NC)]
            for cp in wcps:
                cp.wait()
            return tuple(accs)

        zero = jnp.zeros((_L,), jnp.float32)
        accs = lax.fori_loop(0, NSUBS, p1_body, (zero,) * 14)

        for i in range(14):
            mypart_v[pl.ds(i * _L, _L)] = accs[i]
        pltpu.sync_copy(mypart_v,
                        parts_hbm.at[pl.ds(gtile * 14 * _L, 14 * _L)])

    scratch = (
        [pltpu.VMEM((PT * 3,), jnp.int32)]
        + [pltpu.VMEM((_SUB,), jnp.int32)] * 9
        + [pltpu.VMEM((RUN,), jnp.float32)] * NC
        + [pltpu.VMEM((_SUB,), jnp.float32)] * NC
        + [pltpu.VMEM((14 * _L,), jnp.float32)]
        + [pltpu.VMEM_SHARED((TVP,), jnp.float32)]
        + [pltpu.SemaphoreType.DMA]
        + [pltpu.SemaphoreType.DMA]
    )
    return pl.kernel(
        body,
        out_type=(
            jax.ShapeDtypeStruct((_NCP * F_pad,), jnp.float32),
            jax.ShapeDtypeStruct((_NT * 14 * _L,), jnp.float32),
        ),
        mesh=mesh,
        scratch_types=scratch,
    )


def _make_tc_call(B, F, F_pad):
    BR = 32
    BLKF = BR * 128
    NB = F_pad // BLKF

    def tc_body(nrm_ref, means_ref, out_ref):
        step = pl.program_id(0)

        @pl.when(step == 0)
        def _():
            out_ref[0, 0] = 0.0

        blk = nrm_ref[...]
        gidx = (step * BLKF
                + lax.broadcasted_iota(jnp.int32, (BR, 128), 0) * 128
                + lax.broadcasted_iota(jnp.int32, (BR, 128), 1))
        mask = jnp.where(gidx < F, 1.0, 0.0)
        acc = jnp.zeros((BR, 128), jnp.float32)
        for b in range(B):
            c0 = 3 * b
            dx = blk[c0 + 0] - means_ref[0, c0 + 0]
            dy = blk[c0 + 1] - means_ref[0, c0 + 1]
            dz = blk[c0 + 2] - means_ref[0, c0 + 2]
            acc = acc + jnp.sqrt(dx * dx + dy * dy + dz * dz)
        out_ref[0, 0] += jnp.sum(acc * mask)

    return pl.pallas_call(
        tc_body,
        grid=(NB,),
        in_specs=[
            pl.BlockSpec((_NCP, BR, 128), lambda i: (0, i, 0)),
            pl.BlockSpec(memory_space=pltpu.SMEM),
        ],
        out_specs=pl.BlockSpec(memory_space=pltpu.SMEM),
        out_shape=jax.ShapeDtypeStruct((1, 1), jnp.float32),
    )


def kernel(pred_vertices, gt_vertices, faces):
    del gt_vertices
    B, V, _ = pred_vertices.shape
    F = faces.shape[0]
    PT = -(-F // (_NT * _SUB)) * _SUB
    F_pad = PT * _NT

    TVP = -(-(3 * V * B) // (12 * 128)) * 12 * 128
    tab_flat = jnp.pad(pred_vertices.reshape(B * V * 3),
                       (0, TVP - 3 * V * B))
    faces_t = jnp.pad(faces.astype(jnp.int32).T, ((0, 0), (0, F_pad - F)))

    sc_call = _make_sc_call(V, B, F, F_pad)
    nrm_flat, parts_flat = sc_call(
        tab_flat, faces_t[0], faces_t[1], faces_t[2])

    parts = parts_flat.reshape(_NT, 14, _L)
    nsums = parts[:, :12, :].sum(axis=(0, 2))
    means = jnp.zeros((1, _L), jnp.float32).at[0, :12].set(nsums / F)

    tc_call = _make_tc_call(B, F, F_pad)
    dev_total = tc_call(
        nrm_flat.reshape(_NCP, F_pad // 128, 128), means)[0, 0]

    area_total = parts[:, 12, :].sum()
    aspect_total = parts[:, 13, :].sum()
    return (_W_AREA * area_total + _W_ASPECT * aspect_total
            + _W_DIHEDRAL * dev_total) / (B * F)


# speedup vs baseline: 1.5467x; 1.5467x over previous
import functools

import jax
import jax.numpy as jnp
from jax import lax
from jax.experimental import pallas as pl
from jax.experimental.pallas import tpu as pltpu
from jax.experimental.pallas import tpu_sc as plsc

_W_AREA = 0.1
_W_ASPECT = 0.1
_W_DIHEDRAL = 0.05
_MIN_AREA = 1e-06
_MAX_AR = 10.0

_L = 16
_NCORE = 2
_NS = 16
_NT = _NCORE * _NS
_SUB = 320
_NCP = 16
_TCB = 2048


def _rsqrt(x):
    i = lax.bitcast_convert_type(x, jnp.int32)
    i = 0x5F3759DF - lax.shift_right_logical(i, 1)
    y = lax.bitcast_convert_type(i, jnp.float32)
    for _ in range(3):
        y = y * (1.5 - 0.5 * x * y * y)
    return y


def _sqrt(x):
    return x * _rsqrt(jnp.maximum(x, 1e-36))


def _make_sc_call(V, B, F, F_pad):
    PT = F_pad // _NT
    NSUBS = PT // _SUB
    NC = B * 3
    RUN = 3 * _SUB
    N0, N1 = NSUBS - 1, NSUBS + 1
    PTMAX = N1 * _SUB

    mesh = plsc.VectorSubcoreMesh(
        core_axis_name="c", subcore_axis_name="s", num_cores=_NCORE,
        num_subcores=_NS)

    def body(*refs):
        tabs_hbm = refs[:NC]
        f_hbm = refs[NC:NC + 3]
        nrm_hbm = refs[NC + 3]
        parts_hbm = refs[NC + 4]
        sc = refs[NC + 5:]
        idxall_v = sc[0]
        comp_v = sc[1:1 + NC]
        nstage_v = sc[1 + NC:1 + 2 * NC]
        mypart_v = sc[1 + 2 * NC]
        tab_s = sc[2 + 2 * NC:2 + 3 * NC]
        sem = sc[2 + 3 * NC]
        sem2 = sc[3 + 3 * NC]

        iota = lax.iota(jnp.int32, _L)
        sid = lax.axis_index("s")
        cid = lax.axis_index("c")
        gtile = sid * _NCORE + cid
        ns_me = jnp.where(cid == 0, N0, N1)
        tile_base = jnp.where(cid == 0, sid * N0,
                              _NS * N0 + sid * N1) * _SUB

        for q in range(NC):
            @pl.when(sid == q)
            def _():
                pltpu.sync_copy(tabs_hbm[q], tab_s[q])
        for s in range(3):
            pltpu.sync_copy(f_hbm[s].at[pl.ds(tile_base, PTMAX)],
                            idxall_v.at[pl.ds(s * PTMAX, PTMAX)])
        plsc.subcore_barrier()

        def p1_body(t, carry):
            accs = list(carry)
            sub_base = tile_base + t * _SUB
            cps = [pltpu.async_copy(
                tab_s[q].at[idxall_v.at[pl.ds(s * PTMAX + t * _SUB, _SUB)]],
                comp_v[q].at[pl.ds(s * _SUB, _SUB)], sem)
                for q in range(NC) for s in range(3)]
            for cp in cps:
                cp.wait()

            for g in range(_SUB // _L):
                jbase = g * _L
                gidx = sub_base + jbase + iota
                mask = jnp.where(gidx < F, 1.0, 0.0)
                for b in range(B):
                    c0 = 3 * b
                    (v0x, v0y, v0z), (v1x, v1y, v1z), (v2x, v2y, v2z) = [
                        [comp_v[c0 + c][pl.ds(s * _SUB + jbase, _L)]
                         for c in range(3)] for s in range(3)]
                    ax, ay, az = v1x - v0x, v1y - v0y, v1z - v0z
                    bx, by, bz = v2x - v0x, v2y - v0y, v2z - v0z
                    cx = ay * bz - az * by
                    cy = az * bx - ax * bz
                    cz = ax * by - ay * bx
                    cn2 = cx * cx + cy * cy + cz * cz
                    r = _rsqrt(jnp.maximum(cn2, 1e-36))
                    s_cn = cn2 * r
                    accs[12] = accs[12] + mask * jnp.maximum(
                        _MIN_AREA - 0.5 * s_cn, 0.0)
                    f = jnp.where(s_cn >= 1e-8, r, 1e8)
                    nx, ny, nz = cx * f, cy * f, cz * f
                    accs[c0 + 0] = accs[c0 + 0] + mask * nx
                    accs[c0 + 1] = accs[c0 + 1] + mask * ny
                    accs[c0 + 2] = accs[c0 + 2] + mask * nz
                    nstage_v[c0 + 0][pl.ds(jbase, _L)] = nx
                    nstage_v[c0 + 1][pl.ds(jbase, _L)] = ny
                    nstage_v[c0 + 2][pl.ds(jbase, _L)] = nz
                    l1 = ax * ax + ay * ay + az * az
                    gx, gy, gz = v2x - v1x, v2y - v1y, v2z - v1z
                    l2 = gx * gx + gy * gy + gz * gz
                    l3 = bx * bx + by * by + bz * bz
                    mx2 = jnp.maximum(jnp.maximum(l1, l2), l3)
                    mn2 = jnp.minimum(jnp.minimum(l1, l2), l3)
                    ar = _sqrt(mx2 / jnp.maximum(mn2, 1e-16))
                    accs[13] = accs[13] + mask * jnp.maximum(
                        ar - _MAX_AR, 0.0)
            wcps = [pltpu.async_copy(
                nstage_v[q], nrm_hbm.at[pl.ds(q * F_pad + sub_base, _SUB)],
                sem2) for q in range(NC)]
            for cp in wcps:
                cp.wait()
            return tuple(accs)

        zero = jnp.zeros((_L,), jnp.float32)
        accs = lax.fori_loop(0, ns_me, p1_body, (zero,) * 14)

        for i in range(14):
            mypart_v[pl.ds(i * _L, _L)] = accs[i]
        pltpu.sync_copy(mypart_v,
                        parts_hbm.at[pl.ds(gtile * 14 * _L, 14 * _L)])

    scratch = (
        [pltpu.VMEM((PTMAX * 3,), jnp.int32)]
        + [pltpu.VMEM((RUN,), jnp.float32)] * NC
        + [pltpu.VMEM((_SUB,), jnp.float32)] * NC
        + [pltpu.VMEM((14 * _L,), jnp.float32)]
        + [pltpu.VMEM_SHARED((V,), jnp.float32)] * NC
        + [pltpu.SemaphoreType.DMA]
        + [pltpu.SemaphoreType.DMA]
    )
    return pl.kernel(
        body,
        out_type=(
            jax.ShapeDtypeStruct((_NCP * F_pad,), jnp.float32),
            jax.ShapeDtypeStruct((_NT * 14 * _L,), jnp.float32),
        ),
        mesh=mesh,
        scratch_types=scratch,
    )


def _make_tc_call(B, F, F_pad):
    BR = 32
    BLKF = BR * 128
    NB = F_pad // BLKF

    def tc_body(nrm_ref, means_ref, out_ref):
        step = pl.program_id(0)

        @pl.when(step == 0)
        def _():
            out_ref[0, 0] = 0.0

        blk = nrm_ref[...]
        gidx = (step * BLKF
                + lax.broadcasted_iota(jnp.int32, (BR, 128), 0) * 128
                + lax.broadcasted_iota(jnp.int32, (BR, 128), 1))
        mask = jnp.where(gidx < F, 1.0, 0.0)
        acc = jnp.zeros((BR, 128), jnp.float32)
        for b in range(B):
            c0 = 3 * b
            dx = blk[c0 + 0] - means_ref[0, c0 + 0]
            dy = blk[c0 + 1] - means_ref[0, c0 + 1]
            dz = blk[c0 + 2] - means_ref[0, c0 + 2]
            acc = acc + jnp.sqrt(dx * dx + dy * dy + dz * dz)
        out_ref[0, 0] += jnp.sum(acc * mask)

    return pl.pallas_call(
        tc_body,
        grid=(NB,),
        in_specs=[
            pl.BlockSpec((_NCP, BR, 128), lambda i: (0, i, 0)),
            pl.BlockSpec(memory_space=pltpu.SMEM),
        ],
        out_specs=pl.BlockSpec(memory_space=pltpu.SMEM),
        out_shape=jax.ShapeDtypeStruct((1, 1), jnp.float32),
    )


def kernel(pred_vertices, gt_vertices, faces):
    del gt_vertices
    B, V, _ = pred_vertices.shape
    F = faces.shape[0]
    PT = -(-F // (_NT * _SUB)) * _SUB
    F_pad = PT * _NT

    comp_tabs = pred_vertices.transpose(0, 2, 1).reshape(3 * B, V)
    faces_t = jnp.pad(faces.astype(jnp.int32).T, ((0, 0), (0, F_pad - F)))

    sc_call = _make_sc_call(V, B, F, F_pad)
    nrm_flat, parts_flat = sc_call(
        *[comp_tabs[q] for q in range(3 * B)],
        faces_t[0], faces_t[1], faces_t[2])

    parts = parts_flat.reshape(_NT, 14, _L)
    nsums = parts[:, :12, :].sum(axis=(0, 2))
    means = jnp.zeros((1, _L), jnp.float32).at[0, :12].set(nsums / F)

    tc_call = _make_tc_call(B, F, F_pad)
    dev_total = tc_call(
        nrm_flat.reshape(_NCP, F_pad // 128, 128), means)[0, 0]

    area_total = parts[:, 12, :].sum()
    aspect_total = parts[:, 13, :].sum()
    return (_W_AREA * area_total + _W_ASPECT * aspect_total
            + _W_DIHEDRAL * dev_total) / (B * F)


# speedup vs baseline: 1.6425x; 1.0619x over previous
import functools

import jax
import jax.numpy as jnp
from jax import lax
from jax.experimental import pallas as pl
from jax.experimental.pallas import tpu as pltpu
from jax.experimental.pallas import tpu_sc as plsc

_W_AREA = 0.1
_W_ASPECT = 0.1
_W_DIHEDRAL = 0.05
_MIN_AREA = 1e-06
_MAX_AR = 10.0

_L = 16
_NCORE = 2
_NS = 16
_NT = _NCORE * _NS
_SUB = 320
_NCP = 16
_TCB = 2048


def _rsqrt(x):
    i = lax.bitcast_convert_type(x, jnp.int32)
    i = 0x5F3759DF - lax.shift_right_logical(i, 1)
    y = lax.bitcast_convert_type(i, jnp.float32)
    for _ in range(3):
        y = y * (1.5 - 0.5 * x * y * y)
    return y


def _sqrt(x):
    return x * _rsqrt(jnp.maximum(x, 1e-36))


def _make_sc_call(V, B, F, F_pad):
    PT = F_pad // _NT
    NSUBS = PT // _SUB
    NC = B * 3
    RUN = 3 * _SUB
    N0, N1 = NSUBS + 1, NSUBS - 1
    PTMAX = max(N0, N1) * _SUB

    mesh = plsc.VectorSubcoreMesh(
        core_axis_name="c", subcore_axis_name="s", num_cores=_NCORE,
        num_subcores=_NS)

    def body(*refs):
        tabs_hbm = refs[:NC]
        f_hbm = refs[NC:NC + 3]
        nrm_hbm = refs[NC + 3]
        parts_hbm = refs[NC + 4]
        sc = refs[NC + 5:]
        idxall_v = sc[0]
        comp_v = sc[1:1 + NC]
        nstage_v = sc[1 + NC:1 + 2 * NC]
        mypart_v = sc[1 + 2 * NC]
        tab_s = sc[2 + 2 * NC:2 + 3 * NC]
        sem = sc[2 + 3 * NC]
        sem2 = sc[3 + 3 * NC]

        iota = lax.iota(jnp.int32, _L)
        sid = lax.axis_index("s")
        cid = lax.axis_index("c")
        gtile = sid * _NCORE + cid
        ns_me = jnp.where(cid == 0, N0, N1)
        tile_base = jnp.where(cid == 0, sid * N0,
                              _NS * N0 + sid * N1) * _SUB

        for q in range(NC):
            @pl.when(sid == q)
            def _():
                pltpu.sync_copy(tabs_hbm[q], tab_s[q])
        for s in range(3):
            pltpu.sync_copy(f_hbm[s].at[pl.ds(tile_base, PTMAX)],
                            idxall_v.at[pl.ds(s * PTMAX, PTMAX)])
        plsc.subcore_barrier()

        def p1_body(t, carry):
            accs = list(carry)
            sub_base = tile_base + t * _SUB
            cps = [pltpu.async_copy(
                tab_s[q].at[idxall_v.at[pl.ds(s * PTMAX + t * _SUB, _SUB)]],
                comp_v[q].at[pl.ds(s * _SUB, _SUB)], sem)
                for q in range(NC) for s in range(3)]
            for cp in cps:
                cp.wait()

            for g in range(_SUB // _L):
                jbase = g * _L
                gidx = sub_base + jbase + iota
                mask = jnp.where(gidx < F, 1.0, 0.0)
                for b in range(B):
                    c0 = 3 * b
                    (v0x, v0y, v0z), (v1x, v1y, v1z), (v2x, v2y, v2z) = [
                        [comp_v[c0 + c][pl.ds(s * _SUB + jbase, _L)]
                         for c in range(3)] for s in range(3)]
                    ax, ay, az = v1x - v0x, v1y - v0y, v1z - v0z
                    bx, by, bz = v2x - v0x, v2y - v0y, v2z - v0z
                    cx = ay * bz - az * by
                    cy = az * bx - ax * bz
                    cz = ax * by - ay * bx
                    cn2 = cx * cx + cy * cy + cz * cz
                    r = _rsqrt(jnp.maximum(cn2, 1e-36))
                    s_cn = cn2 * r
                    accs[12] = accs[12] + mask * jnp.maximum(
                        _MIN_AREA - 0.5 * s_cn, 0.0)
                    f = jnp.where(s_cn >= 1e-8, r, 1e8)
                    nx, ny, nz = cx * f, cy * f, cz * f
                    accs[c0 + 0] = accs[c0 + 0] + mask * nx
                    accs[c0 + 1] = accs[c0 + 1] + mask * ny
                    accs[c0 + 2] = accs[c0 + 2] + mask * nz
                    nstage_v[c0 + 0][pl.ds(jbase, _L)] = nx
                    nstage_v[c0 + 1][pl.ds(jbase, _L)] = ny
                    nstage_v[c0 + 2][pl.ds(jbase, _L)] = nz
                    l1 = ax * ax + ay * ay + az * az
                    gx, gy, gz = v2x - v1x, v2y - v1y, v2z - v1z
                    l2 = gx * gx + gy * gy + gz * gz
                    l3 = bx * bx + by * by + bz * bz
                    mx2 = jnp.maximum(jnp.maximum(l1, l2), l3)
                    mn2 = jnp.minimum(jnp.minimum(l1, l2), l3)
                    ar = _sqrt(mx2 / jnp.maximum(mn2, 1e-16))
                    accs[13] = accs[13] + mask * jnp.maximum(
                        ar - _MAX_AR, 0.0)
            wcps = [pltpu.async_copy(
                nstage_v[q], nrm_hbm.at[pl.ds(q * F_pad + sub_base, _SUB)],
                sem2) for q in range(NC)]
            for cp in wcps:
                cp.wait()
            return tuple(accs)

        zero = jnp.zeros((_L,), jnp.float32)
        accs = lax.fori_loop(0, ns_me, p1_body, (zero,) * 14)

        for i in range(14):
            mypart_v[pl.ds(i * _L, _L)] = accs[i]
        pltpu.sync_copy(mypart_v,
                        parts_hbm.at[pl.ds(gtile * 14 * _L, 14 * _L)])

    scratch = (
        [pltpu.VMEM((PTMAX * 3,), jnp.int32)]
        + [pltpu.VMEM((RUN,), jnp.float32)] * NC
        + [pltpu.VMEM((_SUB,), jnp.float32)] * NC
        + [pltpu.VMEM((14 * _L,), jnp.float32)]
        + [pltpu.VMEM_SHARED((V,), jnp.float32)] * NC
        + [pltpu.SemaphoreType.DMA]
        + [pltpu.SemaphoreType.DMA]
    )
    return pl.kernel(
        body,
        out_type=(
            jax.ShapeDtypeStruct((_NCP * F_pad,), jnp.float32),
            jax.ShapeDtypeStruct((_NT * 14 * _L,), jnp.float32),
        ),
        mesh=mesh,
        scratch_types=scratch,
    )


def _make_tc_call(B, F, F_pad):
    BR = 32
    BLKF = BR * 128
    NB = F_pad // BLKF

    def tc_body(nrm_ref, means_ref, out_ref):
        step = pl.program_id(0)

        @pl.when(step == 0)
        def _():
            out_ref[0, 0] = 0.0

        blk = nrm_ref[...]
        gidx = (step * BLKF
                + lax.broadcasted_iota(jnp.int32, (BR, 128), 0) * 128
                + lax.broadcasted_iota(jnp.int32, (BR, 128), 1))
        mask = jnp.where(gidx < F, 1.0, 0.0)
        acc = jnp.zeros((BR, 128), jnp.float32)
        for b in range(B):
            c0 = 3 * b
            dx = blk[c0 + 0] - means_ref[0, c0 + 0]
            dy = blk[c0 + 1] - means_ref[0, c0 + 1]
            dz = blk[c0 + 2] - means_ref[0, c0 + 2]
            acc = acc + jnp.sqrt(dx * dx + dy * dy + dz * dz)
        out_ref[0, 0] += jnp.sum(acc * mask)

    return pl.pallas_call(
        tc_body,
        grid=(NB,),
        in_specs=[
            pl.BlockSpec((_NCP, BR, 128), lambda i: (0, i, 0)),
            pl.BlockSpec(memory_space=pltpu.SMEM),
        ],
        out_specs=pl.BlockSpec(memory_space=pltpu.SMEM),
        out_shape=jax.ShapeDtypeStruct((1, 1), jnp.float32),
    )


def kernel(pred_vertices, gt_vertices, faces):
    del gt_vertices
    B, V, _ = pred_vertices.shape
    F = faces.shape[0]
    PT = -(-F // (_NT * _SUB)) * _SUB
    F_pad = PT * _NT

    comp_tabs = pred_vertices.transpose(0, 2, 1).reshape(3 * B, V)
    faces_t = jnp.pad(faces.astype(jnp.int32).T,
                      ((0, 0), (0, F_pad - F + 2 * _SUB)))

    sc_call = _make_sc_call(V, B, F, F_pad)
    nrm_flat, parts_flat = sc_call(
        *[comp_tabs[q] for q in range(3 * B)],
        faces_t[0], faces_t[1], faces_t[2])

    parts = parts_flat.reshape(_NT, 14, _L)
    nsums = parts[:, :12, :].sum(axis=(0, 2))
    means = jnp.zeros((1, _L), jnp.float32).at[0, :12].set(nsums / F)

    tc_call = _make_tc_call(B, F, F_pad)
    dev_total = tc_call(
        nrm_flat.reshape(_NCP, F_pad // 128, 128), means)[0, 0]

    area_total = parts[:, 12, :].sum()
    aspect_total = parts[:, 13, :].sum()
    return (_W_AREA * area_total + _W_ASPECT * aspect_total
            + _W_DIHEDRAL * dev_total) / (B * F)


# speedup vs baseline: 1.6648x; 1.0136x over previous
import functools

import jax
import jax.numpy as jnp
from jax import lax
from jax.experimental import pallas as pl
from jax.experimental.pallas import tpu as pltpu
from jax.experimental.pallas import tpu_sc as plsc

_W_AREA = 0.1
_W_ASPECT = 0.1
_W_DIHEDRAL = 0.05
_MIN_AREA = 1e-06
_MAX_AR = 10.0

_L = 16
_NCORE = 2
_NS = 16
_NT = _NCORE * _NS
_SUB = 320
_NCP = 16
_TCB = 2048


def _rsqrt(x):
    i = lax.bitcast_convert_type(x, jnp.int32)
    i = 0x5F3759DF - lax.shift_right_logical(i, 1)
    y = lax.bitcast_convert_type(i, jnp.float32)
    for _ in range(3):
        y = y * (1.5 - 0.5 * x * y * y)
    return y


def _sqrt(x):
    return x * _rsqrt(jnp.maximum(x, 1e-36))


def _make_sc_call(V, B, F, F_pad):
    PT = F_pad // _NT
    NSUBS = PT // _SUB
    NC = B * 3
    RUN = 3 * _SUB
    N0, N1 = NSUBS + 1, NSUBS - 1
    PTMAX = max(N0, N1) * _SUB

    mesh = plsc.VectorSubcoreMesh(
        core_axis_name="c", subcore_axis_name="s", num_cores=_NCORE,
        num_subcores=_NS)

    def body(*refs):
        tabs_hbm = refs[:NC]
        f_hbm = refs[NC:NC + 3]
        nrm_hbm = refs[NC + 3]
        parts_hbm = refs[NC + 4]
        sc = refs[NC + 5:]
        idxall_v = sc[0]
        comp_v = sc[1:1 + NC]
        nstage_v = sc[1 + NC:1 + 2 * NC]
        mypart_v = sc[1 + 2 * NC]
        tab_s = sc[2 + 2 * NC:2 + 3 * NC]
        sem = sc[2 + 3 * NC]
        sem2 = sc[3 + 3 * NC]

        iota = lax.iota(jnp.int32, _L)
        sid = lax.axis_index("s")
        cid = lax.axis_index("c")
        gtile = sid * _NCORE + cid
        ns_me = jnp.where(cid == 0, N0, N1)
        tile_base = jnp.where(cid == 0, sid * N0,
                              _NS * N0 + sid * N1) * _SUB

        for q in range(NC):
            @pl.when(sid == q)
            def _():
                pltpu.sync_copy(tabs_hbm[q], tab_s[q])
        for s in range(3):
            pltpu.sync_copy(f_hbm[s].at[pl.ds(tile_base, PTMAX)],
                            idxall_v.at[pl.ds(s * PTMAX, PTMAX)])
        plsc.subcore_barrier()

        def p1_body(t, carry):
            accs = list(carry)
            sub_base = tile_base + t * _SUB
            cps = [pltpu.async_copy(
                tab_s[q].at[idxall_v.at[pl.ds(s * PTMAX + t * _SUB, _SUB)]],
                comp_v[q].at[pl.ds(s * _SUB, _SUB)], sem)
                for q in range(NC) for s in range(3)]
            for cp in cps:
                cp.wait()

            for g in range(_SUB // _L):
                jbase = g * _L
                gidx = sub_base + jbase + iota
                mask = jnp.where(gidx < F, 1.0, 0.0)
                for b in range(B):
                    c0 = 3 * b
                    (v0x, v0y, v0z), (v1x, v1y, v1z), (v2x, v2y, v2z) = [
                        [comp_v[c0 + c][pl.ds(s * _SUB + jbase, _L)]
                         for c in range(3)] for s in range(3)]
                    ax, ay, az = v1x - v0x, v1y - v0y, v1z - v0z
                    bx, by, bz = v2x - v0x, v2y - v0y, v2z - v0z
                    cx = ay * bz - az * by
                    cy = az * bx - ax * bz
                    cz = ax * by - ay * bx
                    cn2 = cx * cx + cy * cy + cz * cz
                    r = _rsqrt(jnp.maximum(cn2, 1e-36))
                    s_cn = cn2 * r
                    accs[12] = accs[12] + mask * jnp.maximum(
                        _MIN_AREA - 0.5 * s_cn, 0.0)
                    f = jnp.where(s_cn >= 1e-8, r, 1e8)
                    nx, ny, nz = cx * f, cy * f, cz * f
                    accs[c0 + 0] = accs[c0 + 0] + mask * nx
                    accs[c0 + 1] = accs[c0 + 1] + mask * ny
                    accs[c0 + 2] = accs[c0 + 2] + mask * nz
                    nstage_v[c0 + 0][pl.ds(jbase, _L)] = nx
                    nstage_v[c0 + 1][pl.ds(jbase, _L)] = ny
                    nstage_v[c0 + 2][pl.ds(jbase, _L)] = nz
                    l1 = ax * ax + ay * ay + az * az
                    gx, gy, gz = v2x - v1x, v2y - v1y, v2z - v1z
                    l2 = gx * gx + gy * gy + gz * gz
                    l3 = bx * bx + by * by + bz * bz
                    mx2 = jnp.maximum(jnp.maximum(l1, l2), l3)
                    mn2 = jnp.minimum(jnp.minimum(l1, l2), l3)
                    ar = _sqrt(mx2 / jnp.maximum(mn2, 1e-16))
                    accs[13] = accs[13] + mask * jnp.maximum(
                        ar - _MAX_AR, 0.0)
            wcps = [pltpu.async_copy(
                nstage_v[q], nrm_hbm.at[pl.ds(q * F_pad + sub_base, _SUB)],
                sem2) for q in range(NC)]
            for cp in wcps:
                cp.wait()
            return tuple(accs)

        zero = jnp.zeros((_L,), jnp.float32)
        accs = lax.fori_loop(0, ns_me, p1_body, (zero,) * 14)

        for i in range(14):
            mypart_v[pl.ds(i * _L, _L)] = accs[i]
        pltpu.sync_copy(mypart_v,
                        parts_hbm.at[pl.ds(gtile * 14 * _L, 14 * _L)])

    scratch = (
        [pltpu.VMEM((PTMAX * 3,), jnp.int32)]
        + [pltpu.VMEM((RUN,), jnp.float32)] * NC
        + [pltpu.VMEM((_SUB,), jnp.float32)] * NC
        + [pltpu.VMEM((14 * _L,), jnp.float32)]
        + [pltpu.VMEM_SHARED((V,), jnp.float32)] * NC
        + [pltpu.SemaphoreType.DMA]
        + [pltpu.SemaphoreType.DMA]
    )
    return pl.kernel(
        body,
        out_type=(
            jax.ShapeDtypeStruct((_NCP * F_pad,), jnp.float32),
            jax.ShapeDtypeStruct((_NT * 14 * _L,), jnp.float32),
        ),
        mesh=mesh,
        scratch_types=scratch,
    )


def _make_tc_call(B, F, F_pad):
    BR = 32
    BLKF = BR * 128
    NB = F_pad // BLKF

    def tc_body(nrm_ref, parts_ref, out_ref, acc_ref):
        step = pl.program_id(0)

        @pl.when(step == 0)
        def _():
            acc_ref[0, 0] = 0.0

        sums = jnp.sum(parts_ref[...], axis=1)
        blk = nrm_ref[...]
        gidx = (step * BLKF
                + lax.broadcasted_iota(jnp.int32, (BR, 128), 0) * 128
                + lax.broadcasted_iota(jnp.int32, (BR, 128), 1))
        mask = jnp.where(gidx < F, 1.0, 0.0)
        acc = jnp.zeros((BR, 128), jnp.float32)
        inv_f = 1.0 / F
        for b in range(B):
            c0 = 3 * b
            dx = blk[c0 + 0] - sums[c0 + 0] * inv_f
            dy = blk[c0 + 1] - sums[c0 + 1] * inv_f
            dz = blk[c0 + 2] - sums[c0 + 2] * inv_f
            acc = acc + jnp.sqrt(dx * dx + dy * dy + dz * dz)
        dev = acc_ref[0, 0] + jnp.sum(acc * mask)
        acc_ref[0, 0] = dev

        @pl.when(step == NB - 1)
        def _():
            out_ref[0, 0] = (_W_AREA * sums[12] + _W_ASPECT * sums[13]
                             + _W_DIHEDRAL * dev) / (B * F)

    return pl.pallas_call(
        tc_body,
        grid=(NB,),
        in_specs=[
            pl.BlockSpec((_NCP, BR, 128), lambda i: (0, i, 0)),
            pl.BlockSpec((16, 32 * _L), lambda i: (0, 0)),
        ],
        out_specs=pl.BlockSpec(memory_space=pltpu.SMEM),
        out_shape=jax.ShapeDtypeStruct((1, 1), jnp.float32),
        scratch_shapes=[pltpu.SMEM((1, 1), jnp.float32)],
    )


def kernel(pred_vertices, gt_vertices, faces):
    del gt_vertices
    B, V, _ = pred_vertices.shape
    F = faces.shape[0]
    PT = -(-F // (_NT * _SUB)) * _SUB
    F_pad = PT * _NT

    comp_tabs = pred_vertices.transpose(0, 2, 1).reshape(3 * B, V)
    faces_t = jnp.pad(faces.astype(jnp.int32).T,
                      ((0, 0), (0, F_pad - F + 2 * _SUB)))

    sc_call = _make_sc_call(V, B, F, F_pad)
    nrm_flat, parts_flat = sc_call(
        *[comp_tabs[q] for q in range(3 * B)],
        faces_t[0], faces_t[1], faces_t[2])

    parts_t = jnp.pad(
        parts_flat.reshape(_NT, 14, _L).transpose(1, 0, 2).reshape(
            14, _NT * _L),
        ((0, 2), (0, 0)))

    tc_call = _make_tc_call(B, F, F_pad)
    return tc_call(nrm_flat.reshape(_NCP, F_pad // 128, 128), parts_t)[0, 0]
